# baseline (device time: 217488 ns/iter reference)
import jax
import jax.numpy as jnp
from jax import lax
from jax.experimental import pallas as pl
from jax.experimental.pallas import tpu as pltpu

N_DEV = 4
SQ = 1024
HQ_LOCAL = 8
DH = 128
DM = 1024
HALF = 512
G32 = 32
BAND = 128
SCALE = 0.08838834764831843
NEG = -1e9
DIMS = (((1,), (1,)), ((), ()))


def _body(x_ref, wq_ref, k_hbm, v_hbm, wo_ref, out_ref,
          k0buf, v0buf, kbnd, vbnd, kstg, vstg,
          q32buf, psend_a, psend_l, precv_a, precv_l,
          osendbuf, obuf,
          copy_sems, kgsems, vgsems,
          q32send, q32recv, k0send, v0send, k0recv, v0recv,
          kbsend, vbsend, kbrecv, vbrecv,
          pasend, parecv, plsend, plrecv,
          osend_sems, orecv_sems, agsend_sems, agrecv_sems):
    my = lax.axis_index("i")

    def rdma(src, dst, ssem, rsem, t):
        return pltpu.make_async_remote_copy(
            src_ref=src, dst_ref=dst, send_sem=ssem, recv_sem=rsem,
            device_id=(t,), device_id_type=pl.DeviceIdType.MESH)

    barrier_sem = pltpu.get_barrier_semaphore()
    for d in (1, 2, 3):
        pl.semaphore_signal(barrier_sem, inc=1,
                            device_id=(lax.rem(my + d, N_DEV),),
                            device_id_type=pl.DeviceIdType.MESH)
    pl.semaphore_wait(barrier_sem, 3)

    @pl.when(my == 0)
    def _():
        pltpu.make_async_copy(k_hbm.at[0, :, pl.ds(0, HQ_LOCAL), :],
                              k0buf, copy_sems.at[0]).start()
        pltpu.make_async_copy(v_hbm.at[0, :, pl.ds(0, HQ_LOCAL), :],
                              v0buf, copy_sems.at[1]).start()
        for t in (1, 2, 3):
            for half in (0, 1):
                sl = pl.ds(half * HALF, HALF)
                rdma(k_hbm.at[0, sl, pl.ds(t * HQ_LOCAL, HQ_LOCAL), :],
                     k0buf.at[sl], k0send.at[(t - 1) * 2 + half],
                     k0recv.at[half], t).start()
                rdma(v_hbm.at[0, sl, pl.ds(t * HQ_LOCAL, HQ_LOCAL), :],
                     v0buf.at[sl], v0send.at[(t - 1) * 2 + half],
                     v0recv.at[half], t).start()

    @pl.when(my == 1)
    def _():
        bsl = pl.ds(0, BAND)
        pltpu.make_async_copy(k_hbm.at[0, bsl, pl.ds(HQ_LOCAL, HQ_LOCAL), :],
                              kbnd, copy_sems.at[2]).start()
        pltpu.make_async_copy(v_hbm.at[0, bsl, pl.ds(HQ_LOCAL, HQ_LOCAL), :],
                              vbnd, copy_sems.at[3]).start()
        for d in (1, 2, 3):
            t = (1 + d) % N_DEV
            rdma(k_hbm.at[0, bsl, pl.ds(t * HQ_LOCAL, HQ_LOCAL), :],
                 kbnd, kbsend.at[d - 1], kbrecv.at[0], t).start()
            rdma(v_hbm.at[0, bsl, pl.ds(t * HQ_LOCAL, HQ_LOCAL), :],
                 vbnd, vbsend.at[d - 1], vbrecv.at[0], t).start()

    q = jnp.dot(x_ref[0], wq_ref[:, :], preferred_element_type=jnp.float32)
    q = (q * SCALE).astype(jnp.bfloat16)

    q32buf[pl.ds(my, 1)] = q[:G32, :][None]
    q32_sends = []
    for d in (1, 2, 3):
        t = lax.rem(my + d, N_DEV)
        s = rdma(q32buf.at[my], q32buf.at[my],
                 q32send.at[d - 1], q32recv.at[d - 1], t)
        s.start()
        q32_sends.append(s)

    def kv_group_dma(g):
        hsl = pl.ds(g * HQ_LOCAL, HQ_LOCAL)
        slot = g % 2
        kc = pltpu.make_async_copy(k_hbm.at[0, :, hsl, :], kstg.at[slot],
                                   kgsems.at[g])
        vc = pltpu.make_async_copy(v_hbm.at[0, :, hsl, :], vstg.at[slot],
                                   vgsems.at[g])
        return kc, vc

    for g in (0, 1):
        for c in kv_group_dma(g):
            c.start()

    for d in (1, 2, 3):
        s = lax.rem(my + (N_DEV - d), N_DEV)
        rdma(q32buf.at[s], q32buf.at[s], q32send.at[0],
             q32recv.at[d - 1], s).wait_recv()

    for g in range(4):
        slot = g % 2
        for c in kv_group_dma(g):
            c.wait()
        q32g = q32buf[g]
        ag_h, lg_h = [], []
        for h in range(HQ_LOCAL):
            qh = q32g[:, h * DH:(h + 1) * DH]
            sm = lax.dot_general(qh, kstg[slot, :, h, :], DIMS,
                                 preferred_element_type=jnp.float32)
            p = jnp.exp(sm)
            lg_h.append(p.sum(axis=1)[:, None])
            ag_h.append(jnp.dot(p.astype(jnp.bfloat16),
                                vstg[slot, :, h, :],
                                preferred_element_type=jnp.float32)[:, None, :])
        if g + 2 < 4:
            for c in kv_group_dma(g + 2):
                c.start()
        psend_a[g] = jnp.concatenate(ag_h, axis=1)
        psend_l[g] = jnp.concatenate(lg_h, axis=1)

    p_sends = []
    for d in (1, 2, 3):
        t = lax.rem(my + d, N_DEV)
        sa = rdma(psend_a.at[t], precv_a.at[d - 1],
                  pasend.at[d - 1], parecv.at[d - 1], t)
        sl_ = rdma(psend_l.at[t], precv_l.at[d - 1],
                   plsend.at[d - 1], plrecv.at[d - 1], t)
        sa.start()
        sl_.start()
        p_sends += [sa, sl_]

    @pl.when(my == 0)
    def _():
        pltpu.make_async_copy(k_hbm.at[0, :, pl.ds(0, HQ_LOCAL), :],
                              k0buf, copy_sems.at[0]).wait()
        pltpu.make_async_copy(v_hbm.at[0, :, pl.ds(0, HQ_LOCAL), :],
                              v0buf, copy_sems.at[1]).wait()

    @pl.when(my != 0)
    def _():
        for half in (0, 1):
            sl = pl.ds(half * HALF, HALF)
            rdma(k0buf.at[sl], k0buf.at[sl], k0send.at[0],
                 k0recv.at[half], my).wait_recv()
            rdma(v0buf.at[sl], v0buf.at[sl], v0send.at[0],
                 v0recv.at[half], my).wait_recv()

    @pl.when(my == 1)
    def _():
        bsl = pl.ds(0, BAND)
        pltpu.make_async_copy(k_hbm.at[0, bsl, pl.ds(HQ_LOCAL, HQ_LOCAL), :],
                              kbnd, copy_sems.at[2]).wait()
        pltpu.make_async_copy(v_hbm.at[0, bsl, pl.ds(HQ_LOCAL, HQ_LOCAL), :],
                              vbnd, copy_sems.at[3]).wait()

    @pl.when(my != 1)
    def _():
        rdma(kbnd, kbnd, kbsend.at[0], kbrecv.at[0], my).wait_recv()
        rdma(vbnd, vbnd, vbsend.at[0], vbrecv.at[0], my).wait_recv()

    def half_bias(half):
        qi = lax.broadcasted_iota(jnp.int32, (SQ, HALF), 0)
        ki = lax.broadcasted_iota(jnp.int32, (SQ, HALF), 1) + half * HALF
        mask = (qi >= G32) & ((jnp.abs(qi - ki) <= 128) | (ki < G32))
        return jnp.where(mask, 0.0, NEG).astype(jnp.float32)

    l0 = [jnp.zeros((SQ,), jnp.float32) for _ in range(HQ_LOCAL)]
    a0 = [jnp.zeros((SQ, DH), jnp.float32) for _ in range(HQ_LOCAL)]
    for half in (0, 1):
        bias = half_bias(half)
        lo, hi = half * HALF, (half + 1) * HALF
        for h in range(HQ_LOCAL):
            qh = q[:, h * DH:(h + 1) * DH]
            sm = lax.dot_general(qh, k0buf[lo:hi, h, :], DIMS,
                                 preferred_element_type=jnp.float32) + bias
            p = jnp.exp(sm)
            l0[h] = l0[h] + p.sum(axis=1)
            a0[h] = a0[h] + jnp.dot(p.astype(jnp.bfloat16),
                                    v0buf[lo:hi, h, :],
                                    preferred_element_type=jnp.float32)

    r = lax.broadcasted_iota(jnp.int32, (BAND, BAND), 0)
    col = lax.broadcasted_iota(jnp.int32, (BAND, BAND), 1)
    bias_tri = jnp.where(r >= col, 0.0, NEG).astype(jnp.float32)
    lb, ab = [], []
    for h in range(HQ_LOCAL):
        qh = q[SQ - BAND:, h * DH:(h + 1) * DH]
        smb = lax.dot_general(qh, kbnd[:, h, :], DIMS,
                              preferred_element_type=jnp.float32) + bias_tri
        pb = jnp.exp(smb)
        lb.append(pb.sum(axis=1))
        ab.append(jnp.dot(pb.astype(jnp.bfloat16), vbnd[:, h, :],
                          preferred_element_type=jnp.float32))

    for d in (1, 2, 3):
        rdma(precv_a.at[d - 1], precv_a.at[d - 1], pasend.at[0],
             parecv.at[d - 1], my).wait_recv()
        rdma(precv_l.at[d - 1], precv_l.at[d - 1], plsend.at[0],
             plrecv.at[d - 1], my).wait_recv()
    a32 = precv_a[0] + precv_a[1] + precv_a[2]
    l32 = precv_l[0] + precv_l[1] + precv_l[2]
    for g in range(4):
        ind = jnp.where(my == g, 1.0, 0.0).astype(jnp.float32)
        a32 = a32 + psend_a[g] * ind
        l32 = l32 + psend_l[g] * ind

    ctx_heads = []
    for h in range(HQ_LOCAL):
        l_h = jnp.concatenate(
            [l32[:, h], l0[h][G32:SQ - BAND], l0[h][SQ - BAND:] + lb[h]])
        acc_h = jnp.concatenate(
            [a32[:, h, :], a0[h][G32:SQ - BAND], a0[h][SQ - BAND:] + ab[h]],
            axis=0)
        ctx_heads.append((acc_h / l_h[:, None]).astype(jnp.bfloat16))
    ctx = jnp.concatenate(ctx_heads, axis=1)
    partial = jnp.dot(ctx, wo_ref[:, :], preferred_element_type=jnp.float32)

    QR = SQ // N_DEV
    osendbuf[:, :] = partial.astype(jnp.bfloat16)
    rs_sends = []
    for d in (1, 2, 3):
        t = lax.rem(my + d, N_DEV)
        o = rdma(osendbuf.at[pl.ds(t * QR, QR), :], obuf.at[d - 1],
                 osend_sems.at[d - 1], orecv_sems.at[d - 1], t)
        o.start()
        rs_sends.append(o)
    for d in (1, 2, 3):
        rdma(obuf.at[d - 1], obuf.at[d - 1], osend_sems.at[0],
             orecv_sems.at[d - 1], my).wait_recv()

    own_q = osendbuf[pl.ds(my * QR, QR), :].astype(jnp.float32)
    reduced = (own_q + obuf[0].astype(jnp.float32)
               + obuf[1].astype(jnp.float32) + obuf[2].astype(jnp.float32))
    out_ref[0, pl.ds(my * QR, QR), :] = reduced

    ag_sends = []
    for d in (1, 2, 3):
        t = lax.rem(my + d, N_DEV)
        o = rdma(out_ref.at[0, pl.ds(my * QR, QR), :],
                 out_ref.at[0, pl.ds(my * QR, QR), :],
                 agsend_sems.at[d - 1], agrecv_sems.at[d - 1], t)
        o.start()
        ag_sends.append(o)
    for d in (1, 2, 3):
        s = lax.rem(my + (N_DEV - d), N_DEV)
        rdma(out_ref.at[0, pl.ds(s * QR, QR), :],
             out_ref.at[0, pl.ds(s * QR, QR), :],
             agsend_sems.at[0], agrecv_sems.at[d - 1], s).wait_recv()

    for snd in q32_sends + p_sends + rs_sends + ag_sends:
        snd.wait_send()

    @pl.when(my == 0)
    def _():
        for t in (1, 2, 3):
            for half in (0, 1):
                sl = pl.ds(half * HALF, HALF)
                rdma(k_hbm.at[0, sl, pl.ds(t * HQ_LOCAL, HQ_LOCAL), :],
                     k0buf.at[sl], k0send.at[(t - 1) * 2 + half],
                     k0recv.at[half], t).wait_send()
                rdma(v_hbm.at[0, sl, pl.ds(t * HQ_LOCAL, HQ_LOCAL), :],
                     v0buf.at[sl], v0send.at[(t - 1) * 2 + half],
                     v0recv.at[half], t).wait_send()

    @pl.when(my == 1)
    def _():
        bsl = pl.ds(0, BAND)
        for d in (1, 2, 3):
            t = (1 + d) % N_DEV
            rdma(k_hbm.at[0, bsl, pl.ds(t * HQ_LOCAL, HQ_LOCAL), :],
                 kbnd, kbsend.at[d - 1], kbrecv.at[0], t).wait_send()
            rdma(v_hbm.at[0, bsl, pl.ds(t * HQ_LOCAL, HQ_LOCAL), :],
                 vbnd, vbsend.at[d - 1], vbrecv.at[0], t).wait_send()


def kernel(x, Wq, K_ext, V_ext, Wo):
    xb = x.astype(jnp.bfloat16)
    wqb = Wq.astype(jnp.bfloat16)
    kb = K_ext.astype(jnp.bfloat16)
    vb = V_ext.astype(jnp.bfloat16)
    wob = Wo.astype(jnp.bfloat16)

    return pl.pallas_call(
        _body,
        out_shape=jax.ShapeDtypeStruct((1, SQ, DM), jnp.float32),
        in_specs=[
            pl.BlockSpec(memory_space=pltpu.MemorySpace.VMEM),
            pl.BlockSpec(memory_space=pltpu.MemorySpace.VMEM),
            pl.BlockSpec(memory_space=pltpu.MemorySpace.HBM),
            pl.BlockSpec(memory_space=pltpu.MemorySpace.HBM),
            pl.BlockSpec(memory_space=pltpu.MemorySpace.VMEM),
        ],
        out_specs=pl.BlockSpec(memory_space=pltpu.MemorySpace.VMEM),
        scratch_shapes=[
            pltpu.VMEM((SQ, HQ_LOCAL, DH), jnp.bfloat16),
            pltpu.VMEM((SQ, HQ_LOCAL, DH), jnp.bfloat16),
            pltpu.VMEM((BAND, HQ_LOCAL, DH), jnp.bfloat16),
            pltpu.VMEM((BAND, HQ_LOCAL, DH), jnp.bfloat16),
            pltpu.VMEM((2, SQ, HQ_LOCAL, DH), jnp.bfloat16),
            pltpu.VMEM((2, SQ, HQ_LOCAL, DH), jnp.bfloat16),
            pltpu.VMEM((N_DEV, G32, DM), jnp.bfloat16),
            pltpu.VMEM((N_DEV, G32, HQ_LOCAL, DH), jnp.float32),
            pltpu.VMEM((N_DEV, G32, HQ_LOCAL), jnp.float32),
            pltpu.VMEM((3, G32, HQ_LOCAL, DH), jnp.float32),
            pltpu.VMEM((3, G32, HQ_LOCAL), jnp.float32),
            pltpu.VMEM((SQ, DM), jnp.bfloat16),
            pltpu.VMEM((3, SQ // N_DEV, DM), jnp.bfloat16),
            pltpu.SemaphoreType.DMA((4,)),
            pltpu.SemaphoreType.DMA((4,)),
            pltpu.SemaphoreType.DMA((4,)),
            pltpu.SemaphoreType.DMA((3,)),
            pltpu.SemaphoreType.DMA((3,)),
            pltpu.SemaphoreType.DMA((6,)),
            pltpu.SemaphoreType.DMA((6,)),
            pltpu.SemaphoreType.DMA((2,)),
            pltpu.SemaphoreType.DMA((2,)),
            pltpu.SemaphoreType.DMA((3,)),
            pltpu.SemaphoreType.DMA((3,)),
            pltpu.SemaphoreType.DMA((1,)),
            pltpu.SemaphoreType.DMA((1,)),
            pltpu.SemaphoreType.DMA((3,)),
            pltpu.SemaphoreType.DMA((3,)),
            pltpu.SemaphoreType.DMA((3,)),
            pltpu.SemaphoreType.DMA((3,)),
            pltpu.SemaphoreType.DMA((3,)),
            pltpu.SemaphoreType.DMA((3,)),
            pltpu.SemaphoreType.DMA((3,)),
            pltpu.SemaphoreType.DMA((3,)),
        ],
        compiler_params=pltpu.CompilerParams(
            collective_id=0, vmem_limit_bytes=63 * 1024 * 1024),
    )(xb, wqb, kb, vb, wob)
